# baseline (speedup 1.0000x reference)
import functools

import jax
import jax.numpy as jnp
from jax import lax
from jax.experimental import pallas as pl
from jax.experimental.pallas import tpu as pltpu
from jax.experimental.pallas import tpu_sc as plsc

N = 10000
E = 320000
D = 128

NC = 2
NS = 16
K = 128

NP = 10240
RW = NP // NS
EP = 323584
EPT = EP // (NC * NS)
CHUNKS = EPT // K
RB = 1280


UD = 8
UD_IT, UD_TAIL = CHUNKS // UD, CHUNKS % UD
UM = 2
UM_IT, UM_TAIL = CHUNKS // UM, CHUNKS % UM


def _sc_deg_body(dst_hbm, z_hbm, out_hbm, idx_v, ones_v, acc):
    c = lax.axis_index("c")
    s = lax.axis_index("s")

    def fill(i, _):
        ones_v[i, :] = jnp.ones((16,), jnp.float32)
        return 0

    lax.fori_loop(0, K, fill, 0)

    @pl.when(s == 0)
    def _():
        pltpu.sync_copy(z_hbm, acc)

    plsc.subcore_barrier()

    ebase = (c * NS + s) * EPT

    def chunk(k, _):
        pltpu.sync_copy(dst_hbm.at[pl.ds(ebase + k * K, K)], idx_v)
        pltpu.sync_copy(ones_v, acc.at[idx_v], add=True)
        return 0

    lax.fori_loop(0, CHUNKS, chunk, 0)
    plsc.subcore_barrier()

    @pl.when(s == 0)
    def _():
        pltpu.sync_copy(acc, out_hbm.at[c])


def _sc_msg_body(g_hbm, ei_hbm, z_hbm, out_hbm, idx_v, rows_v, acc,
                 isem, gsem, ssem):
    c = lax.axis_index("c")
    s = lax.axis_index("s")

    @pl.when(s == 0)
    def _():
        pltpu.sync_copy(z_hbm, acc)

    plsc.subcore_barrier()

    ebase = (c * NS + s) * EPT

    def batch(e, n):
        loads = [
            pltpu.async_copy(ei_hbm.at[:, pl.ds(e + j * K, K)], idx_v.at[j], isem)
            for j in range(n)
        ]
        for d in loads:
            d.wait()
        gathers = [
            pltpu.async_copy(g_hbm.at[idx_v.at[j, 0]], rows_v.at[j], gsem)
            for j in range(n)
        ]
        for d in gathers:
            d.wait()
        for j in range(n):
            pltpu.sync_copy(rows_v.at[j], acc.at[idx_v.at[j, 1]], add=True)

    def it(m, _):
        batch(ebase + m * UM * K, UM)
        return 0

    lax.fori_loop(0, UM_IT, it, 0)
    if UM_TAIL:
        batch(ebase + UM_IT * UM * K, UM_TAIL)
    plsc.subcore_barrier()

    @pl.when(s == 0)
    def _():
        pltpu.sync_copy(acc, out_hbm.at[c])


@functools.lru_cache(maxsize=None)
def _sc_kernels():
    mesh = plsc.VectorSubcoreMesh(
        core_axis_name="c", subcore_axis_name="s", num_cores=NC, num_subcores=NS
    )
    sc_deg = pl.kernel(
        _sc_deg_body,
        out_type=jax.ShapeDtypeStruct((NC, NP, 16), jnp.float32),
        mesh=mesh,
        scratch_types=[
            pltpu.VMEM((K,), jnp.int32),
            pltpu.VMEM((K, 16), jnp.float32),
            pltpu.VMEM_SHARED((NP, 16), jnp.float32),
        ],
    )
    sc_msg = pl.kernel(
        _sc_msg_body,
        out_type=jax.ShapeDtypeStruct((NC, NP, D), jnp.float32),
        mesh=mesh,
        scratch_types=[
            pltpu.VMEM((UM, 2, K), jnp.int32),
            pltpu.VMEM((UM, K, D), jnp.float32),
            pltpu.VMEM_SHARED((NP, D), jnp.float32),
            pltpu.SemaphoreType.DMA,
            pltpu.SemaphoreType.DMA,
            pltpu.SemaphoreType.DMA,
        ],
    )
    return sc_deg, sc_msg


def _dinv(deg_ref):
    return lax.rsqrt(deg_ref[0, :, 0:1] + deg_ref[1, :, 0:1] + 1.0)


def _tc_a_body(x_ref, w_ref, deg_ref, g_ref):
    g_ref[...] = jnp.dot(
        x_ref[...], w_ref[...], preferred_element_type=jnp.float32
    ) * _dinv(deg_ref)


def _tc_b_body(p_ref, g1_ref, deg_ref, b_ref, w_ref, g2_ref):
    dinv = _dinv(deg_ref)
    h = (p_ref[0] + p_ref[1] + g1_ref[...]) * dinv + b_ref[...]
    g2_ref[...] = jnp.dot(
        h, w_ref[...], preferred_element_type=jnp.float32
    ) * dinv


def _tc_c_body(p_ref, g2_ref, deg_ref, b_ref, o_ref):
    o_ref[...] = (p_ref[0] + p_ref[1] + g2_ref[...]) * _dinv(deg_ref) + b_ref[...]


_row_spec = pl.BlockSpec((RB, D), lambda i: (i, 0))
_w_spec = pl.BlockSpec((D, D), lambda i: (0, 0))
_deg_spec = pl.BlockSpec((2, RB, 16), lambda i: (0, i, 0))
_p_spec = pl.BlockSpec((2, RB, D), lambda i: (0, i, 0))
_b_spec = pl.BlockSpec((1, D), lambda i: (0, 0))
_GRID = (NP // RB,)
_row_out = jax.ShapeDtypeStruct((NP, D), jnp.float32)

_tc_a = pl.pallas_call(
    _tc_a_body,
    grid=_GRID,
    in_specs=[_row_spec, _w_spec, _deg_spec],
    out_specs=_row_spec,
    out_shape=_row_out,
)

_tc_b = pl.pallas_call(
    _tc_b_body,
    grid=_GRID,
    in_specs=[_p_spec, _row_spec, _deg_spec, _b_spec, _w_spec],
    out_specs=_row_spec,
    out_shape=_row_out,
)

_tc_c = pl.pallas_call(
    _tc_c_body,
    grid=_GRID,
    in_specs=[_p_spec, _row_spec, _deg_spec, _b_spec],
    out_specs=_row_spec,
    out_shape=_row_out,
)


def kernel(x, edge_index, W1, b1, W2, b2):
    pad = N + (jnp.arange(EP - E, dtype=jnp.int32) % (NP - N))
    ei = jnp.concatenate([edge_index, jnp.stack([pad, pad])], axis=1)
    dst = jnp.concatenate([edge_index[1], pad])
    x_p = jnp.pad(x, ((0, NP - N), (0, 0)))
    z16 = jnp.zeros((NP, 16), jnp.float32)
    zD = jnp.zeros((NP, D), jnp.float32)

    sc_deg, sc_msg = _sc_kernels()
    degp = sc_deg(dst, z16)
    g1 = _tc_a(x_p, W1, degp)
    p1 = sc_msg(g1, ei, zD)
    g2 = _tc_b(p1, g1, degp, b1.reshape(1, D), W2)
    p2 = sc_msg(g2, ei, zD)
    out = _tc_c(p2, g2, degp, b2.reshape(1, D))
    return out[:N]


# speedup vs baseline: 1.0720x; 1.0720x over previous
import functools

import jax
import jax.numpy as jnp
from jax import lax
from jax.experimental import pallas as pl
from jax.experimental.pallas import tpu as pltpu
from jax.experimental.pallas import tpu_sc as plsc

N = 10000
E = 320000
D = 128

NC = 2
NS = 16
K = 128

NP = 10240
RW = NP // NS
EP = 323584
EPT = EP // (NC * NS)
CHUNKS = EPT // K
RB = 1280


UD = 8
UD_IT, UD_TAIL = CHUNKS // UD, CHUNKS % UD
UM = 2
UM_IT, UM_TAIL = CHUNKS // UM, CHUNKS % UM


def _sc_deg_body(dst_hbm, z_hbm, out_hbm, idx_v, ones_v, acc, isem):
    c = lax.axis_index("c")
    s = lax.axis_index("s")

    def fill(i, _):
        ones_v[i, :] = jnp.ones((16,), jnp.float32)
        return 0

    lax.fori_loop(0, K, fill, 0)

    @pl.when(s == 0)
    def _():
        pltpu.sync_copy(z_hbm, acc)

    plsc.subcore_barrier()

    ebase = (c * NS + s) * EPT

    def batch(e, n):
        loads = [
            pltpu.async_copy(dst_hbm.at[pl.ds(e + j * K, K)], idx_v.at[j], isem)
            for j in range(n)
        ]
        for d in loads:
            d.wait()
        for j in range(n):
            pltpu.sync_copy(ones_v, acc.at[idx_v.at[j]], add=True)

    def it(m, _):
        batch(ebase + m * UD * K, UD)
        return 0

    lax.fori_loop(0, UD_IT, it, 0)
    if UD_TAIL:
        batch(ebase + UD_IT * UD * K, UD_TAIL)
    plsc.subcore_barrier()

    @pl.when(s == 0)
    def _():
        pltpu.sync_copy(acc, out_hbm.at[c])


def _sc_msg_body(g_hbm, ei_hbm, z_hbm, out_hbm, idx_v, rows_v, acc,
                 isem, gsem, ssem):
    c = lax.axis_index("c")
    s = lax.axis_index("s")

    @pl.when(s == 0)
    def _():
        pltpu.sync_copy(z_hbm, acc)

    plsc.subcore_barrier()

    ebase = (c * NS + s) * EPT

    def batch(e, n):
        loads = [
            pltpu.async_copy(ei_hbm.at[:, pl.ds(e + j * K, K)], idx_v.at[j], isem)
            for j in range(n)
        ]
        for d in loads:
            d.wait()
        gathers = [
            pltpu.async_copy(g_hbm.at[idx_v.at[j, 0]], rows_v.at[j], gsem)
            for j in range(n)
        ]
        for d in gathers:
            d.wait()
        for j in range(n):
            pltpu.sync_copy(rows_v.at[j], acc.at[idx_v.at[j, 1]], add=True)

    def it(m, _):
        batch(ebase + m * UM * K, UM)
        return 0

    lax.fori_loop(0, UM_IT, it, 0)
    if UM_TAIL:
        batch(ebase + UM_IT * UM * K, UM_TAIL)
    plsc.subcore_barrier()

    @pl.when(s == 0)
    def _():
        pltpu.sync_copy(acc, out_hbm.at[c])


@functools.lru_cache(maxsize=None)
def _sc_kernels():
    mesh = plsc.VectorSubcoreMesh(
        core_axis_name="c", subcore_axis_name="s", num_cores=NC, num_subcores=NS
    )
    sc_deg = pl.kernel(
        _sc_deg_body,
        out_type=jax.ShapeDtypeStruct((NC, NP, 16), jnp.float32),
        mesh=mesh,
        scratch_types=[
            pltpu.VMEM((UD, K), jnp.int32),
            pltpu.VMEM((K, 16), jnp.float32),
            pltpu.VMEM_SHARED((NP, 16), jnp.float32),
            pltpu.SemaphoreType.DMA,
        ],
    )
    sc_msg = pl.kernel(
        _sc_msg_body,
        out_type=jax.ShapeDtypeStruct((NC, NP, D), jnp.float32),
        mesh=mesh,
        scratch_types=[
            pltpu.VMEM((UM, 2, K), jnp.int32),
            pltpu.VMEM((UM, K, D), jnp.float32),
            pltpu.VMEM_SHARED((NP, D), jnp.float32),
            pltpu.SemaphoreType.DMA,
            pltpu.SemaphoreType.DMA,
            pltpu.SemaphoreType.DMA,
        ],
    )
    return sc_deg, sc_msg


def _dinv(deg_ref):
    return lax.rsqrt(deg_ref[0, :, 0:1] + deg_ref[1, :, 0:1] + 1.0)


def _tc_m_body(x_ref, w_ref, h_ref):
    h_ref[...] = jnp.dot(x_ref[...], w_ref[...], preferred_element_type=jnp.float32)


def _tc_s_body(h_ref, deg_ref, g_ref):
    g_ref[...] = h_ref[...] * _dinv(deg_ref)


def _tc_b_body(p_ref, g1_ref, deg_ref, b_ref, w_ref, g2_ref):
    dinv = _dinv(deg_ref)
    h = (p_ref[0] + p_ref[1] + g1_ref[...]) * dinv + b_ref[...]
    g2_ref[...] = jnp.dot(
        h, w_ref[...], preferred_element_type=jnp.float32
    ) * dinv


def _tc_c_body(p_ref, g2_ref, deg_ref, b_ref, o_ref):
    o_ref[...] = (p_ref[0] + p_ref[1] + g2_ref[...]) * _dinv(deg_ref) + b_ref[...]


_row_spec = pl.BlockSpec((RB, D), lambda i: (i, 0))
_w_spec = pl.BlockSpec((D, D), lambda i: (0, 0))
_deg_spec = pl.BlockSpec((2, RB, 16), lambda i: (0, i, 0))
_p_spec = pl.BlockSpec((2, RB, D), lambda i: (0, i, 0))
_b_spec = pl.BlockSpec((1, D), lambda i: (0, 0))
_GRID = (NP // RB,)
_row_out = jax.ShapeDtypeStruct((NP, D), jnp.float32)

_tc_m = pl.pallas_call(
    _tc_m_body,
    grid=_GRID,
    in_specs=[_row_spec, _w_spec],
    out_specs=_row_spec,
    out_shape=_row_out,
)

_tc_s = pl.pallas_call(
    _tc_s_body,
    grid=_GRID,
    in_specs=[_row_spec, _deg_spec],
    out_specs=_row_spec,
    out_shape=_row_out,
)

_tc_b = pl.pallas_call(
    _tc_b_body,
    grid=_GRID,
    in_specs=[_p_spec, _row_spec, _deg_spec, _b_spec, _w_spec],
    out_specs=_row_spec,
    out_shape=_row_out,
)

_tc_c = pl.pallas_call(
    _tc_c_body,
    grid=_GRID,
    in_specs=[_p_spec, _row_spec, _deg_spec, _b_spec],
    out_specs=_row_spec,
    out_shape=_row_out,
)


def kernel(x, edge_index, W1, b1, W2, b2):
    pad = N + (jnp.arange(EP - E, dtype=jnp.int32) % (NP - N))
    ei = jnp.concatenate([edge_index, jnp.stack([pad, pad])], axis=1)
    dst = jnp.concatenate([edge_index[1], pad])
    x_p = jnp.pad(x, ((0, NP - N), (0, 0)))
    z16 = jnp.zeros((NP, 16), jnp.float32)
    zD = jnp.zeros((NP, D), jnp.float32)

    sc_deg, sc_msg = _sc_kernels()
    degp = sc_deg(dst, z16)
    h1 = _tc_m(x_p, W1)
    g1 = _tc_s(h1, degp)
    p1 = sc_msg(g1, ei, zD)
    g2 = _tc_b(p1, g1, degp, b1.reshape(1, D), W2)
    p2 = sc_msg(g2, ei, zD)
    out = _tc_c(p2, g2, degp, b2.reshape(1, D))
    return out[:N]


# speedup vs baseline: 1.1489x; 1.0717x over previous
import functools

import jax
import jax.numpy as jnp
from jax import lax
from jax.experimental import pallas as pl
from jax.experimental.pallas import tpu as pltpu
from jax.experimental.pallas import tpu_sc as plsc

N = 10000
E = 320000
D = 128

NC = 2
NS = 16
K = 128

NP = 10240
RW = NP // NS
EP = 323584
EPT = EP // (NC * NS)
CHUNKS = EPT // K
RB = 1280


UD = 8
UD_IT, UD_TAIL = CHUNKS // UD, CHUNKS % UD
UM = 2
UM_IT, UM_TAIL = CHUNKS // UM, CHUNKS % UM


def _sc_deg_body(dst_hbm, z_hbm, out_hbm, idx_v, ones_v, acc, isem):
    c = lax.axis_index("c")
    s = lax.axis_index("s")

    def fill(i, _):
        ones_v[i, :] = jnp.ones((16,), jnp.float32)
        return 0

    lax.fori_loop(0, K, fill, 0)

    @pl.when(s == 0)
    def _():
        pltpu.sync_copy(z_hbm, acc)

    plsc.subcore_barrier()

    ebase = (c * NS + s) * EPT

    def batch(e, n):
        loads = [
            pltpu.async_copy(dst_hbm.at[pl.ds(e + j * K, K)], idx_v.at[j], isem)
            for j in range(n)
        ]
        for d in loads:
            d.wait()
        for j in range(n):
            pltpu.sync_copy(ones_v, acc.at[idx_v.at[j]], add=True)

    def it(m, _):
        batch(ebase + m * UD * K, UD)
        return 0

    lax.fori_loop(0, UD_IT, it, 0)
    if UD_TAIL:
        batch(ebase + UD_IT * UD * K, UD_TAIL)
    plsc.subcore_barrier()

    @pl.when(s == 0)
    def _():
        pltpu.sync_copy(acc, out_hbm.at[c])


SB = 16
SB_IT, SB_TAIL = CHUNKS // SB, CHUNKS % SB


def _sc_msg_body(g_hbm, ei_hbm, z_hbm, out_hbm, idx_v, rows_v, acc, gsem):
    c = lax.axis_index("c")
    s = lax.axis_index("s")

    @pl.when(s == 0)
    def _():
        pltpu.sync_copy(z_hbm, acc)

    plsc.subcore_barrier()

    chbase = (c * NS + s) * CHUNKS

    def superbatch(q0, nch):
        pltpu.sync_copy(ei_hbm.at[pl.ds(q0, nch)], idx_v.at[pl.ds(0, nch)])
        for p in range(0, nch - 1, 2):
            g0 = pltpu.async_copy(g_hbm.at[idx_v.at[p, 0]], rows_v.at[0], gsem)
            g1 = pltpu.async_copy(g_hbm.at[idx_v.at[p + 1, 0]], rows_v.at[1], gsem)
            g0.wait()
            g1.wait()
            pltpu.sync_copy(rows_v.at[0], acc.at[idx_v.at[p, 1]], add=True)
            pltpu.sync_copy(rows_v.at[1], acc.at[idx_v.at[p + 1, 1]], add=True)
        if nch % 2:
            j = nch - 1
            pltpu.async_copy(g_hbm.at[idx_v.at[j, 0]], rows_v.at[0], gsem).wait()
            pltpu.sync_copy(rows_v.at[0], acc.at[idx_v.at[j, 1]], add=True)

    def it(q, _):
        superbatch(chbase + q * SB, SB)
        return 0

    lax.fori_loop(0, SB_IT, it, 0)
    if SB_TAIL:
        superbatch(chbase + SB_IT * SB, SB_TAIL)
    plsc.subcore_barrier()

    @pl.when(s == 0)
    def _():
        pltpu.sync_copy(acc, out_hbm.at[c])


@functools.lru_cache(maxsize=None)
def _sc_kernels():
    mesh = plsc.VectorSubcoreMesh(
        core_axis_name="c", subcore_axis_name="s", num_cores=NC, num_subcores=NS
    )
    sc_deg = pl.kernel(
        _sc_deg_body,
        out_type=jax.ShapeDtypeStruct((NC, NP, 16), jnp.float32),
        mesh=mesh,
        scratch_types=[
            pltpu.VMEM((UD, K), jnp.int32),
            pltpu.VMEM((K, 16), jnp.float32),
            pltpu.VMEM_SHARED((NP, 16), jnp.float32),
            pltpu.SemaphoreType.DMA,
        ],
    )
    sc_msg = pl.kernel(
        _sc_msg_body,
        out_type=jax.ShapeDtypeStruct((NC, NP, D), jnp.float32),
        mesh=mesh,
        scratch_types=[
            pltpu.VMEM((SB, 2, K), jnp.int32),
            pltpu.VMEM((2, K, D), jnp.float32),
            pltpu.VMEM_SHARED((NP, D), jnp.float32),
            pltpu.SemaphoreType.DMA,
        ],
    )
    return sc_deg, sc_msg


def _dinv(deg_ref):
    return lax.rsqrt(deg_ref[0, :, 0:1] + deg_ref[1, :, 0:1] + 1.0)


def _tc_m_body(x_ref, w_ref, h_ref):
    h_ref[...] = jnp.dot(x_ref[...], w_ref[...], preferred_element_type=jnp.float32)


def _tc_s_body(h_ref, deg_ref, g_ref):
    g_ref[...] = h_ref[...] * _dinv(deg_ref)


def _tc_b_body(p_ref, g1_ref, deg_ref, b_ref, w_ref, g2_ref):
    dinv = _dinv(deg_ref)
    h = (p_ref[0] + p_ref[1] + g1_ref[...]) * dinv + b_ref[...]
    g2_ref[...] = jnp.dot(
        h, w_ref[...], preferred_element_type=jnp.float32
    ) * dinv


def _tc_c_body(p_ref, g2_ref, deg_ref, b_ref, o_ref):
    o_ref[...] = (p_ref[0] + p_ref[1] + g2_ref[...]) * _dinv(deg_ref) + b_ref[...]


_row_spec = pl.BlockSpec((RB, D), lambda i: (i, 0))
_w_spec = pl.BlockSpec((D, D), lambda i: (0, 0))
_deg_spec = pl.BlockSpec((2, RB, 16), lambda i: (0, i, 0))
_p_spec = pl.BlockSpec((2, RB, D), lambda i: (0, i, 0))
_b_spec = pl.BlockSpec((1, D), lambda i: (0, 0))
_GRID = (NP // RB,)
_row_out = jax.ShapeDtypeStruct((NP, D), jnp.float32)

_tc_m = pl.pallas_call(
    _tc_m_body,
    grid=_GRID,
    in_specs=[_row_spec, _w_spec],
    out_specs=_row_spec,
    out_shape=_row_out,
)

_tc_s = pl.pallas_call(
    _tc_s_body,
    grid=_GRID,
    in_specs=[_row_spec, _deg_spec],
    out_specs=_row_spec,
    out_shape=_row_out,
)

_tc_b = pl.pallas_call(
    _tc_b_body,
    grid=_GRID,
    in_specs=[_p_spec, _row_spec, _deg_spec, _b_spec, _w_spec],
    out_specs=_row_spec,
    out_shape=_row_out,
)

_tc_c = pl.pallas_call(
    _tc_c_body,
    grid=_GRID,
    in_specs=[_p_spec, _row_spec, _deg_spec, _b_spec],
    out_specs=_row_spec,
    out_shape=_row_out,
)


def kernel(x, edge_index, W1, b1, W2, b2):
    pad = N + (jnp.arange(EP - E, dtype=jnp.int32) % (NP - N))
    ei = jnp.concatenate([edge_index, jnp.stack([pad, pad])], axis=1)
    ei_ch = ei.reshape(2, EP // K, K).transpose(1, 0, 2)
    dst = jnp.concatenate([edge_index[1], pad])
    x_p = jnp.pad(x, ((0, NP - N), (0, 0)))
    z16 = jnp.zeros((NP, 16), jnp.float32)
    zD = jnp.zeros((NP, D), jnp.float32)

    sc_deg, sc_msg = _sc_kernels()
    degp = sc_deg(dst, z16)
    h1 = _tc_m(x_p, W1)
    g1 = _tc_s(h1, degp)
    p1 = sc_msg(g1, ei_ch, zD)
    g2 = _tc_b(p1, g1, degp, b1.reshape(1, D), W2)
    p2 = sc_msg(g2, ei_ch, zD)
    out = _tc_c(p2, g2, degp, b2.reshape(1, D))
    return out[:N]


# speedup vs baseline: 1.4403x; 1.2536x over previous
import functools

import jax
import jax.numpy as jnp
from jax import lax
from jax.experimental import pallas as pl
from jax.experimental.pallas import tpu as pltpu
from jax.experimental.pallas import tpu_sc as plsc

N = 10000
E = 320000
D = 128

NC = 2
NS = 16
K = 128

NP = 10240
RW = NP // NS
EP = 323584
EPT = EP // (NC * NS)
CHUNKS = EPT // K
RB = 1280


UD = 8
UD_IT, UD_TAIL = CHUNKS // UD, CHUNKS % UD
UM = 2
UM_IT, UM_TAIL = CHUNKS // UM, CHUNKS % UM


def _sc_deg_body(dst_hbm, z_hbm, out_hbm, idx_v, ones_v, acc, isem):
    c = lax.axis_index("c")
    s = lax.axis_index("s")

    def fill(i, _):
        ones_v[i, :] = jnp.ones((16,), jnp.float32)
        return 0

    lax.fori_loop(0, K, fill, 0)

    @pl.when(s == 0)
    def _():
        pltpu.sync_copy(z_hbm, acc)

    plsc.subcore_barrier()

    ebase = (c * NS + s) * EPT

    def batch(e, n):
        loads = [
            pltpu.async_copy(dst_hbm.at[pl.ds(e + j * K, K)], idx_v.at[j], isem)
            for j in range(n)
        ]
        for d in loads:
            d.wait()
        for j in range(n):
            pltpu.sync_copy(ones_v, acc.at[idx_v.at[j]], add=True)

    def it(m, _):
        batch(ebase + m * UD * K, UD)
        return 0

    lax.fori_loop(0, UD_IT, it, 0)
    if UD_TAIL:
        batch(ebase + UD_IT * UD * K, UD_TAIL)
    plsc.subcore_barrier()

    @pl.when(s == 0)
    def _():
        pltpu.sync_copy(acc, out_hbm.at[c])


SB = 16
SB_IT, SB_TAIL = CHUNKS // SB, CHUNKS % SB


def _sc_msg_body(g_hbm, ei_hbm, z_hbm, out_hbm, idx_v, rows_v, acc, gsem):
    c = lax.axis_index("c")
    s = lax.axis_index("s")

    @pl.when(s == 0)
    def _():
        pltpu.sync_copy(z_hbm, acc)

    plsc.subcore_barrier()

    chbase = (c * NS + s) * CHUNKS

    def superbatch(q0, nch):
        pltpu.sync_copy(ei_hbm.at[pl.ds(q0, nch)], idx_v.at[pl.ds(0, nch)])
        pend = pltpu.async_copy(g_hbm.at[idx_v.at[0, 0]], rows_v.at[0], gsem)
        for p in range(nch):
            if p + 1 < nch:
                nxt = pltpu.async_copy(
                    g_hbm.at[idx_v.at[p + 1, 0]], rows_v.at[(p + 1) % 2], gsem
                )
            pend.wait()
            pltpu.sync_copy(rows_v.at[p % 2], acc.at[idx_v.at[p, 1]], add=True)
            if p + 1 < nch:
                pend = nxt

    def it(q, _):
        superbatch(chbase + q * SB, SB)
        return 0

    lax.fori_loop(0, SB_IT, it, 0)
    if SB_TAIL:
        superbatch(chbase + SB_IT * SB, SB_TAIL)
    plsc.subcore_barrier()

    @pl.when(s == 0)
    def _():
        pltpu.sync_copy(acc, out_hbm.at[c])


@functools.lru_cache(maxsize=None)
def _sc_kernels():
    mesh = plsc.VectorSubcoreMesh(
        core_axis_name="c", subcore_axis_name="s", num_cores=NC, num_subcores=NS
    )
    sc_deg = pl.kernel(
        _sc_deg_body,
        out_type=jax.ShapeDtypeStruct((NC, NP, 16), jnp.float32),
        mesh=mesh,
        scratch_types=[
            pltpu.VMEM((UD, K), jnp.int32),
            pltpu.VMEM((K, 16), jnp.float32),
            pltpu.VMEM_SHARED((NP, 16), jnp.float32),
            pltpu.SemaphoreType.DMA,
        ],
    )
    sc_msg = pl.kernel(
        _sc_msg_body,
        out_type=jax.ShapeDtypeStruct((NC, NP, D), jnp.float32),
        mesh=mesh,
        scratch_types=[
            pltpu.VMEM((SB, 2, K), jnp.int32),
            pltpu.VMEM((2, K, D), jnp.float32),
            pltpu.VMEM_SHARED((NP, D), jnp.float32),
            pltpu.SemaphoreType.DMA,
        ],
    )
    return sc_deg, sc_msg


def _dinv(deg_ref):
    return lax.rsqrt(deg_ref[0, :, 0:1] + deg_ref[1, :, 0:1] + 1.0)


def _tc_m_body(x_ref, w_ref, h_ref):
    h_ref[...] = jnp.dot(x_ref[...], w_ref[...], preferred_element_type=jnp.float32)


def _tc_s_body(h_ref, deg_ref, g_ref):
    g_ref[...] = h_ref[...] * _dinv(deg_ref)


def _tc_b_body(p_ref, g1_ref, deg_ref, b_ref, w_ref, g2_ref):
    dinv = _dinv(deg_ref)
    h = (p_ref[0] + p_ref[1] + g1_ref[...]) * dinv + b_ref[...]
    g2_ref[...] = jnp.dot(
        h, w_ref[...], preferred_element_type=jnp.float32
    ) * dinv


def _tc_c_body(p_ref, g2_ref, deg_ref, b_ref, o_ref):
    o_ref[...] = (p_ref[0] + p_ref[1] + g2_ref[...]) * _dinv(deg_ref) + b_ref[...]


_row_spec = pl.BlockSpec((RB, D), lambda i: (i, 0))
_w_spec = pl.BlockSpec((D, D), lambda i: (0, 0))
_deg_spec = pl.BlockSpec((2, RB, 16), lambda i: (0, i, 0))
_p_spec = pl.BlockSpec((2, RB, D), lambda i: (0, i, 0))
_b_spec = pl.BlockSpec((1, D), lambda i: (0, 0))
_GRID = (NP // RB,)
_row_out = jax.ShapeDtypeStruct((NP, D), jnp.float32)

_tc_m = pl.pallas_call(
    _tc_m_body,
    grid=_GRID,
    in_specs=[_row_spec, _w_spec],
    out_specs=_row_spec,
    out_shape=_row_out,
)

_tc_s = pl.pallas_call(
    _tc_s_body,
    grid=_GRID,
    in_specs=[_row_spec, _deg_spec],
    out_specs=_row_spec,
    out_shape=_row_out,
)

_tc_b = pl.pallas_call(
    _tc_b_body,
    grid=_GRID,
    in_specs=[_p_spec, _row_spec, _deg_spec, _b_spec, _w_spec],
    out_specs=_row_spec,
    out_shape=_row_out,
)

_tc_c = pl.pallas_call(
    _tc_c_body,
    grid=_GRID,
    in_specs=[_p_spec, _row_spec, _deg_spec, _b_spec],
    out_specs=_row_spec,
    out_shape=_row_out,
)


def kernel(x, edge_index, W1, b1, W2, b2):
    pad = N + (jnp.arange(EP - E, dtype=jnp.int32) % (NP - N))
    ei = jnp.concatenate([edge_index, jnp.stack([pad, pad])], axis=1)
    ei_ch = ei.reshape(2, EP // K, K).transpose(1, 0, 2)
    dst = jnp.concatenate([edge_index[1], pad])
    x_p = jnp.pad(x, ((0, NP - N), (0, 0)))
    z16 = jnp.zeros((NP, 16), jnp.float32)
    zD = jnp.zeros((NP, D), jnp.float32)

    sc_deg, sc_msg = _sc_kernels()
    degp = sc_deg(dst, z16)
    h1 = _tc_m(x_p, W1)
    g1 = _tc_s(h1, degp)
    p1 = sc_msg(g1, ei_ch, zD)
    g2 = _tc_b(p1, g1, degp, b1.reshape(1, D), W2)
    p2 = sc_msg(g2, ei_ch, zD)
    out = _tc_c(p2, g2, degp, b2.reshape(1, D))
    return out[:N]
